# Initial kernel scaffold; baseline (speedup 1.0000x reference)
import jax
import jax.numpy as jnp
from jax.experimental import pallas as pl


def kernel(orig_weight, new_weight, input):
    raise NotImplementedError("write your pallas kernel here")


# speedup vs baseline: 1.4770x; 1.4770x over previous
import functools

import jax
import jax.numpy as jnp
from jax import lax
from jax.experimental import pallas as pl
from jax.experimental.pallas import tpu as pltpu
from jax.experimental.pallas import tpu_sc as plsc

VOCAB = 1000000
NUM_PROMPT = 128
D = 32
TOTAL = 4096 * 200

NC, NS, L = 2, 16, 16
NW = NC * NS
PER_W = TOTAL // NW
BLK = 1024
NBLK = PER_W // BLK
GPER = 128
NG = BLK // GPER


def _body(orig_hbm, new_hbm, idx_hbm, out_hbm,
          new_tab_v, idx_v, safe_v, rows_v, sem):
    wid = lax.axis_index("s") * NC + lax.axis_index("c")

    pltpu.sync_copy(new_hbm, new_tab_v)

    def block(b, carry):
        base = wid * PER_W + b * BLK
        pltpu.sync_copy(idx_hbm.at[pl.ds(base, BLK)], idx_v)

        def clamp(t, c):
            v = idx_v[pl.ds(t * L, L)]
            safe_v[pl.ds(t * L, L)] = jnp.minimum(v, VOCAB - 1)
            return c
        lax.fori_loop(0, BLK // L, clamp, 0)

        descs = [
            pltpu.async_copy(
                orig_hbm.at[safe_v.at[pl.ds(j * GPER, GPER)]],
                rows_v.at[pl.ds(j * GPER, GPER)],
                sem,
            )
            for j in range(NG)
        ]
        for dsc in descs:
            dsc.wait()

        def fix(t, c):
            v = idx_v[pl.ds(t * L, L)]
            cnt = plsc.all_reduce_population_count(v >= VOCAB)

            @pl.when(cnt[0] > 0)
            def _():
                mask = v >= VOCAB
                pidx = jnp.maximum(v - VOCAB, 0)
                rowid = t * L + lax.iota(jnp.int32, L)
                for dd in range(D):
                    dvec = jnp.full((L,), dd, jnp.int32)
                    vals = plsc.load_gather(new_tab_v, [pidx, dvec])
                    plsc.store_scatter(rows_v, [rowid, dvec], vals, mask=mask)
            return c
        lax.fori_loop(0, BLK // L, fix, 0)

        pltpu.sync_copy(rows_v, out_hbm.at[pl.ds(base, BLK)])
        return carry

    lax.fori_loop(0, NBLK, block, 0)


@functools.partial(jax.jit, static_argnames=())
def _lookup(orig_weight, new_weight, idx_flat):
    mesh = plsc.VectorSubcoreMesh(core_axis_name="c", subcore_axis_name="s")
    f = pl.kernel(
        _body,
        out_type=jax.ShapeDtypeStruct((TOTAL, D), jnp.float32),
        mesh=mesh,
        scratch_types=[
            pltpu.VMEM((NUM_PROMPT, D), jnp.float32),
            pltpu.VMEM((BLK,), jnp.int32),
            pltpu.VMEM((BLK,), jnp.int32),
            pltpu.VMEM((BLK, D), jnp.float32),
            pltpu.SemaphoreType.DMA,
        ],
        compiler_params=pltpu.CompilerParams(
            needs_layout_passes=False, use_tc_tiling_on_sc=False),
    )
    return f(orig_weight, new_weight, idx_flat)


def kernel(orig_weight, new_weight, input):
    idx_flat = input.reshape(-1).astype(jnp.int32)
    out = _lookup(orig_weight, new_weight, idx_flat)
    return out.reshape(input.shape + (D,))
